# Initial kernel scaffold; baseline (speedup 1.0000x reference)
import functools

import jax
import jax.numpy as jnp
from jax import lax
from jax.experimental import pallas as pl
from jax.experimental.pallas import tpu as pltpu
from jax.experimental.pallas import tpu_sc as plsc

B = 4096
T = 200
D = 32
N = B * T
NW = 32
PER_W = N // NW
CS = 1600
NCH = PER_W // CS
BLKS = CS // T


def _sc_body(x_hbm, tok_hbm, pos_hbm, col_hbm, out_hbm,
             idx0, idx1, rows0, rows1, add_v, col_v,
             s_i0, s_i1, s_g0, s_g1, s_s0, s_s1):
    wid = lax.axis_index("s") * 2 + lax.axis_index("c")
    base = wid * PER_W

    pltpu.sync_copy(pos_hbm, add_v)
    pltpu.sync_copy(col_hbm, col_v)
    for f in range(10):
        c0 = col_v[f, pl.ds(0, 16)]
        c1 = col_v[f, pl.ds(16, 16)]

        def _acc(i, carry, f=f, c0=c0, c1=c1):
            t = f * 20 + i
            add_v[t, pl.ds(0, 16)] = add_v[t, pl.ds(0, 16)] + c0
            add_v[t, pl.ds(16, 16)] = add_v[t, pl.ds(16, 16)] + c1
            return carry

        lax.fori_loop(0, 20, _acc, 0)

    idx = (idx0, idx1)
    rows = (rows0, rows1)
    s_i = (s_i0, s_i1)
    s_g = (s_g0, s_g1)
    s_s = (s_s0, s_s1)

    def start_idx(g):
        return pltpu.async_copy(
            x_hbm.at[pl.ds(base + g * CS, CS)], idx[g & 1], s_i[g & 1])

    def start_gather(g):
        return pltpu.async_copy(tok_hbm.at[idx[g & 1]], rows[g & 1], s_g[g & 1])

    def start_scatter(g):
        return pltpu.async_copy(
            rows[g & 1], out_hbm.at[pl.ds(base + g * CS, CS)], s_s[g & 1])

    idx_h = {}
    gat_h = {}
    sct_h = {}

    idx_h[0] = start_idx(0)
    idx_h[0].wait()
    gat_h[0] = start_gather(0)
    if NCH > 1:
        idx_h[1] = start_idx(1)

    for g in range(NCH):
        b = g & 1
        gat_h[g].wait()
        if g + 1 < NCH:
            idx_h[g + 1].wait()
            if g >= 1:
                sct_h[g - 1].wait()
            gat_h[g + 1] = start_gather(g + 1)
            if g + 2 < NCH:
                idx_h[g + 2] = start_idx(g + 2)

        def _add(t, carry, b=b):
            a0 = add_v[t, pl.ds(0, 16)]
            a1 = add_v[t, pl.ds(16, 16)]
            for blk in range(BLKS):
                r = blk * T + t
                rows[b][r, pl.ds(0, 16)] = rows[b][r, pl.ds(0, 16)] + a0
                rows[b][r, pl.ds(16, 16)] = rows[b][r, pl.ds(16, 16)] + a1
            return carry

        lax.fori_loop(0, T, _add, 0)
        sct_h[g] = start_scatter(g)

    sct_h[NCH - 2].wait()
    sct_h[NCH - 1].wait()


@jax.jit
def _sc_kernel(x_flat, token_table, pos_table, col_table):
    mesh = plsc.VectorSubcoreMesh(core_axis_name="c", subcore_axis_name="s")
    f = pl.kernel(
        _sc_body,
        mesh=mesh,
        out_type=jax.ShapeDtypeStruct((N, D), jnp.float32),
        scratch_types=[
            pltpu.VMEM((CS,), jnp.int32),
            pltpu.VMEM((CS,), jnp.int32),
            pltpu.VMEM((CS, D), jnp.float32),
            pltpu.VMEM((CS, D), jnp.float32),
            pltpu.VMEM((T, D), jnp.float32),
            pltpu.VMEM((11, D), jnp.float32),
            pltpu.SemaphoreType.DMA,
            pltpu.SemaphoreType.DMA,
            pltpu.SemaphoreType.DMA,
            pltpu.SemaphoreType.DMA,
            pltpu.SemaphoreType.DMA,
            pltpu.SemaphoreType.DMA,
        ],
    )
    return f(x_flat, token_table, pos_table, col_table)


def kernel(x, token_table, pos_table, col_table):
    x_flat = x.reshape(N).astype(jnp.int32)
    out = _sc_kernel(x_flat, token_table, pos_table, col_table)
    return out.reshape(B, T, D)


# speedup vs baseline: 1.4909x; 1.4909x over previous
import functools

import jax
import jax.numpy as jnp
from jax import lax
from jax.experimental import pallas as pl
from jax.experimental.pallas import tpu as pltpu
from jax.experimental.pallas import tpu_sc as plsc

B = 4096
T = 200
D = 32
N = B * T
NW = 32
PER_W = N // NW
CS = 1600
NCH = PER_W // CS
BLKS = CS // T


def _sc_body(x_hbm, tok_hbm, pos_hbm, col_hbm, out_hbm,
             idx0, idx1, rows0, rows1, add_v, col_v,
             s_i0, s_i1, s_g0, s_g1, s_s0, s_s1):
    wid = lax.axis_index("s") * 2 + lax.axis_index("c")
    base = wid * PER_W

    pltpu.sync_copy(pos_hbm, add_v)
    pltpu.sync_copy(col_hbm, col_v)
    for f in range(10):
        c0 = col_v[f, pl.ds(0, 16)]
        c1 = col_v[f, pl.ds(16, 16)]

        def _acc(i, carry, f=f, c0=c0, c1=c1):
            t = f * 20 + i
            add_v[t, pl.ds(0, 16)] = add_v[t, pl.ds(0, 16)] + c0
            add_v[t, pl.ds(16, 16)] = add_v[t, pl.ds(16, 16)] + c1
            return carry

        lax.fori_loop(0, 20, _acc, 0)

    idx = (idx0, idx1)
    rows = (rows0, rows1)
    s_i = (s_i0, s_i1)
    s_g = (s_g0, s_g1)
    s_s = (s_s0, s_s1)

    def start_idx(g):
        return pltpu.async_copy(
            x_hbm.at[pl.ds(base + g * CS, CS)], idx[g & 1], s_i[g & 1])

    def start_gather(g):
        return pltpu.async_copy(tok_hbm.at[idx[g & 1]], rows[g & 1], s_g[g & 1])

    def start_scatter(g):
        return pltpu.async_copy(
            rows[g & 1], out_hbm.at[pl.ds(base + g * CS, CS)], s_s[g & 1])

    idx_h = {}
    gat_h = {}
    sct_h = {}

    idx_h[0] = start_idx(0)
    idx_h[0].wait()
    gat_h[0] = start_gather(0)
    if NCH > 1:
        idx_h[1] = start_idx(1)

    for g in range(NCH):
        b = g & 1
        gat_h[g].wait()
        if g + 1 < NCH:
            idx_h[g + 1].wait()
            if g >= 1:
                sct_h[g - 1].wait()
            gat_h[g + 1] = start_gather(g + 1)
            if g + 2 < NCH:
                idx_h[g + 2] = start_idx(g + 2)

        def _add(t, carry, b=b):
            a0 = add_v[t, pl.ds(0, 16)]
            a1 = add_v[t, pl.ds(16, 16)]
            for blk in range(BLKS):
                r = blk * T + t
                rows[b][r, pl.ds(0, 16)] = rows[b][r, pl.ds(0, 16)] + a0
                rows[b][r, pl.ds(16, 16)] = rows[b][r, pl.ds(16, 16)] + a1
            return carry

        lax.fori_loop(0, T, _add, 0)
        sct_h[g] = start_scatter(g)

    sct_h[NCH - 2].wait()
    sct_h[NCH - 1].wait()


@jax.jit
def _sc_kernel(x_flat, token_table, pos_table, col_table):
    mesh = plsc.VectorSubcoreMesh(core_axis_name="c", subcore_axis_name="s")
    f = pl.kernel(
        _sc_body,
        mesh=mesh,
        out_type=jax.ShapeDtypeStruct((N, D), jnp.float32),
        compiler_params=pltpu.CompilerParams(use_tc_tiling_on_sc=False),
        scratch_types=[
            pltpu.VMEM((CS,), jnp.int32),
            pltpu.VMEM((CS,), jnp.int32),
            pltpu.VMEM((CS, D), jnp.float32),
            pltpu.VMEM((CS, D), jnp.float32),
            pltpu.VMEM((T, D), jnp.float32),
            pltpu.VMEM((11, D), jnp.float32),
            pltpu.SemaphoreType.DMA,
            pltpu.SemaphoreType.DMA,
            pltpu.SemaphoreType.DMA,
            pltpu.SemaphoreType.DMA,
            pltpu.SemaphoreType.DMA,
            pltpu.SemaphoreType.DMA,
        ],
    )
    return f(x_flat, token_table, pos_table, col_table)


def kernel(x, token_table, pos_table, col_table):
    x_flat = x.reshape(N).astype(jnp.int32)
    out = _sc_kernel(x_flat, token_table, pos_table, col_table)
    return out.reshape(B, T, D)


# speedup vs baseline: 1.8886x; 1.2667x over previous
import jax
import jax.numpy as jnp
from jax import lax
from jax.experimental import pallas as pl
from jax.experimental.pallas import tpu as pltpu
from jax.experimental.pallas import tpu_sc as plsc

B = 4096
T = 200
D = 32
NW = 32
CB = 512
NBB = B // CB
NTASK = T * NBB
PER_W = NTASK // NW
PADW = CB + 1


def _sc_body(xt_hbm, tok_hbm, post_hbm, col_hbm, out_hbm,
             idx0, idx1, rows0, rows1, outt0, outt1, post_v, col_v,
             s_i0, s_i1, s_g0, s_g1, s_s0, s_s1):
    wid = lax.axis_index("s") * 2 + lax.axis_index("c")
    base_task = wid * PER_W

    pltpu.sync_copy(post_hbm, post_v)
    pltpu.sync_copy(col_hbm, col_v)

    iota16 = lax.iota(jnp.int32, 16)
    iota16b = iota16 + 16

    idx = (idx0, idx1)
    rows = (rows0, rows1)
    outt = (outt0, outt1)
    s_i = (s_i0, s_i1)
    s_g = (s_g0, s_g1)
    s_s = (s_s0, s_s1)

    def task_tb(i):
        tk = base_task + i
        return tk >> 3, pl.multiple_of((tk & 7) << 9, CB)

    def start_idx(i):
        t, b0 = task_tb(i)
        return pltpu.async_copy(
            xt_hbm.at[t, pl.ds(b0, CB)], idx[i & 1], s_i[i & 1])

    def start_gather(i):
        return pltpu.async_copy(tok_hbm.at[idx[i & 1]], rows[i & 1], s_g[i & 1])

    def start_scatter(i):
        t, b0 = task_tb(i)
        return pltpu.async_copy(
            outt[i & 1].at[:, pl.ds(0, CB)],
            out_hbm.at[t, :, pl.ds(b0, CB)], s_s[i & 1])

    idx_h = {}
    gat_h = {}
    sct_h = {}

    idx_h[0] = start_idx(0)
    idx_h[0].wait()
    gat_h[0] = start_gather(0)
    if PER_W > 1:
        idx_h[1] = start_idx(1)

    for i in range(PER_W):
        p = i & 1
        gat_h[i].wait()
        if i + 1 < PER_W:
            if i + 2 < PER_W:
                idx_h[i + 2] = start_idx(i + 2)
            idx_h[i + 1].wait()
            if i >= 1:
                sct_h[i - 1].wait()
            gat_h[i + 1] = start_gather(i + 1)

        t, _ = task_tb(i)
        f = (t * 3277) >> 16
        tspl = jnp.full((16,), t, jnp.int32)
        fspl = jnp.full((16,), f, jnp.int32)
        a0 = (plsc.load_gather(post_v, [iota16, tspl])
              + plsc.load_gather(col_v, [fspl, iota16]))
        a1 = (plsc.load_gather(post_v, [iota16b, tspl])
              + plsc.load_gather(col_v, [fspl, iota16b]))

        def _tr(j, carry, p=p, a0=a0, a1=a1):
            v0 = rows[p][j, pl.ds(0, 16)] + a0
            v1 = rows[p][j, pl.ds(16, 16)] + a1
            jspl = jnp.full((16,), 0, jnp.int32) + j
            plsc.store_scatter(outt[p], [iota16, jspl], v0)
            plsc.store_scatter(outt[p], [iota16b, jspl], v1)
            return carry

        lax.fori_loop(0, CB, _tr, 0)
        sct_h[i] = start_scatter(i)

    sct_h[PER_W - 2].wait()
    sct_h[PER_W - 1].wait()


@jax.jit
def _sc_kernel(xt, token_table, post, col_table):
    mesh = plsc.VectorSubcoreMesh(core_axis_name="c", subcore_axis_name="s")
    f = pl.kernel(
        _sc_body,
        mesh=mesh,
        out_type=jax.ShapeDtypeStruct((T, D, B), jnp.float32),
        compiler_params=pltpu.CompilerParams(
            use_tc_tiling_on_sc=False, needs_layout_passes=False),
        scratch_types=[
            pltpu.VMEM((CB,), jnp.int32),
            pltpu.VMEM((CB,), jnp.int32),
            pltpu.VMEM((CB, D), jnp.float32),
            pltpu.VMEM((CB, D), jnp.float32),
            pltpu.VMEM((D, PADW), jnp.float32),
            pltpu.VMEM((D, PADW), jnp.float32),
            pltpu.VMEM((D, T), jnp.float32),
            pltpu.VMEM((11, D), jnp.float32),
            pltpu.SemaphoreType.DMA,
            pltpu.SemaphoreType.DMA,
            pltpu.SemaphoreType.DMA,
            pltpu.SemaphoreType.DMA,
            pltpu.SemaphoreType.DMA,
            pltpu.SemaphoreType.DMA,
        ],
    )
    return f(xt, token_table, post, col_table)


def kernel(x, token_table, pos_table, col_table):
    xt = x.T.astype(jnp.int32)
    post = pos_table.T
    outt = _sc_kernel(xt, token_table, post, col_table)
    return outt.transpose(2, 0, 1)
